# baseline (speedup 1.0000x reference)
import jax
import jax.numpy as jnp
from jax import lax
from jax.experimental import pallas as pl
from jax.experimental.pallas import tpu as pltpu
from jax.experimental.pallas import tpu_sc as plsc

N = 10000
E = 320000
F_IN = 128
H = 8
C1 = 8
HC = H * C1
LW = 16
G1W = HC + LW
NC = 40
NCP = 48
HNCP = H * NCP
G2W = HNCP + LW
D2W = 2 * LW
NEG = 0.2

NP_ = 10240
NW = 32
NCHUNK = 168
NCH2 = NCHUNK + 2
B = 64
BL = 128
NCHUNKL = 84
NCH2L = NCHUNKL + 2
EP = NW * NCHUNK * B
RPS = NP_ // 16
BLK = 1024
GRID = NP_ // BLK

_SC_PARAMS = pltpu.CompilerParams(use_tc_tiling_on_sc=False,
                                  needs_layout_passes=False)


def _tc1_body(x_ref, w1_ref, as_ref, ad_ref, g_out, ad_out):
    h = jnp.dot(x_ref[...], w1_ref[...], preferred_element_type=jnp.float32)
    as16 = jnp.dot(h, as_ref[...], preferred_element_type=jnp.float32)
    g_out[...] = jnp.concatenate([h, as16], axis=1)
    ad_out[...] = jnp.dot(h, ad_ref[...], preferred_element_type=jnp.float32)


def _tc1(xp, W1, As1, Ad1):
    return pl.pallas_call(
        _tc1_body,
        grid=(GRID,),
        in_specs=[
            pl.BlockSpec((BLK, F_IN), lambda i: (i, 0)),
            pl.BlockSpec((F_IN, HC), lambda i: (0, 0)),
            pl.BlockSpec((HC, LW), lambda i: (0, 0)),
            pl.BlockSpec((HC, LW), lambda i: (0, 0)),
        ],
        out_specs=[
            pl.BlockSpec((BLK, G1W), lambda i: (i, 0)),
            pl.BlockSpec((BLK, LW), lambda i: (i, 0)),
        ],
        out_shape=[
            jax.ShapeDtypeStruct((NP_, G1W), jnp.float32),
            jax.ShapeDtypeStruct((NP_, LW), jnp.float32),
        ],
    )(xp, W1, As1, Ad1)


def _tc2_body(acc_ref, e16_ref, b1_ref, w2p_ref, as2_ref, ad2_ref,
              g_out, as_out, ad_out):
    num = acc_ref[0][:, :HC] + acc_ref[1][:, :HC]
    den16 = acc_ref[0][:, HC:] + acc_ref[1][:, HC:]
    den = jnp.dot(den16, e16_ref[...], preferred_element_type=jnp.float32)
    out1 = num / (den + 1e-16)
    v = out1 + b1_ref[...]
    h1e = jnp.where(v > 0, v, jnp.exp(v) - 1.0)
    h2 = jnp.dot(h1e, w2p_ref[...], preferred_element_type=jnp.float32)
    as2 = jnp.dot(h2, as2_ref[...], preferred_element_type=jnp.float32)
    g_out[...] = jnp.concatenate([h2, as2], axis=1)
    as_out[...] = as2
    ad_out[...] = jnp.dot(h2, ad2_ref[...], preferred_element_type=jnp.float32)


def _tc2(acc1, E16, b1r, W2p, As2M, Ad2M):
    return pl.pallas_call(
        _tc2_body,
        grid=(GRID,),
        in_specs=[
            pl.BlockSpec((2, BLK, G1W), lambda i: (0, i, 0)),
            pl.BlockSpec((LW, HC), lambda i: (0, 0)),
            pl.BlockSpec((1, HC), lambda i: (0, 0)),
            pl.BlockSpec((HC, HNCP), lambda i: (0, 0)),
            pl.BlockSpec((HNCP, LW), lambda i: (0, 0)),
            pl.BlockSpec((HNCP, LW), lambda i: (0, 0)),
        ],
        out_specs=[
            pl.BlockSpec((BLK, G2W), lambda i: (i, 0)),
            pl.BlockSpec((BLK, LW), lambda i: (i, 0)),
            pl.BlockSpec((BLK, LW), lambda i: (i, 0)),
        ],
        out_shape=[
            jax.ShapeDtypeStruct((NP_, G2W), jnp.float32),
            jax.ShapeDtypeStruct((NP_, LW), jnp.float32),
            jax.ShapeDtypeStruct((NP_, LW), jnp.float32),
        ],
    )(acc1, E16, b1r, W2p, As2M, Ad2M)


def _tc2c_body(den_ref, ad_ref, d_out):
    r = (1.0 / H) / (den_ref[0] + den_ref[1] + 1e-16)
    d_out[...] = jnp.concatenate([ad_ref[...], r], axis=1)


def _tc2c(den2p, ad2s):
    return pl.pallas_call(
        _tc2c_body,
        grid=(GRID,),
        in_specs=[
            pl.BlockSpec((2, BLK, LW), lambda i: (0, i, 0)),
            pl.BlockSpec((BLK, LW), lambda i: (i, 0)),
        ],
        out_specs=pl.BlockSpec((BLK, D2W), lambda i: (i, 0)),
        out_shape=jax.ShapeDtypeStruct((NP_, D2W), jnp.float32),
    )(den2p, ad2s)


def _tc3_body(num_ref, b2_ref, out_ref):
    z = num_ref[0] + num_ref[1] + b2_ref[...]
    col = lax.broadcasted_iota(jnp.int32, (BLK, NCP), 1)
    mask = col < NC
    zm = jnp.where(mask, z, -1e30)
    m = jnp.max(zm, axis=1, keepdims=True)
    s = jnp.sum(jnp.where(mask, jnp.exp(zm - m), 0.0), axis=1, keepdims=True)
    out_ref[...] = (z - m) - jnp.log(s)


def _tc3(num2, b2p):
    return pl.pallas_call(
        _tc3_body,
        grid=(GRID,),
        in_specs=[
            pl.BlockSpec((2, BLK, NCP), lambda i: (0, i, 0)),
            pl.BlockSpec((1, NCP), lambda i: (0, 0)),
        ],
        out_specs=pl.BlockSpec((BLK, NCP), lambda i: (i, 0)),
        out_shape=jax.ShapeDtypeStruct((NP_, NCP), jnp.float32),
    )(num2, b2p)


def _stage_idx(srcs16, dsts16, wid, s16, d16):
    pltpu.sync_copy(srcs16.at[wid], s16)
    pltpu.sync_copy(dsts16.at[wid], d16)


def _unpack_idx(s16, d16, src32, dst32, j, slot):
    for ref16, ref32 in ((s16, src32), (d16, dst32)):
        for t in range(ref16.shape[1] // 32):
            va, vb = plsc.unpack(
                ref16[j, pl.ds(32 * t, 32)],
                format=plsc.PackFormat.INTERLEAVED,
                preferred_element_type=jnp.int32,
            )
            ref32[slot, pl.ds(32 * t, 16)] = va
            ref32[slot, pl.ds(32 * t + 16, 16)] = vb


def _sc1_body(G1, AD1, srcs16, dsts16, z80,
              out_acc,
              s16, d16, src32, dst32, g0, d0, g1, d1, mw,
              acc, sg0, sd0, sg1, sd1):
    cid = lax.axis_index("c")
    sid = lax.axis_index("s")
    wid = cid * 16 + sid

    _stage_idx(srcs16, dsts16, wid, s16, d16)
    r0 = sid * RPS
    pltpu.sync_copy(z80.at[pl.ds(r0, RPS)], acc.at[pl.ds(r0, RPS)])
    plsc.subcore_barrier()

    bufs = ((g0, d0, sg0, sd0), (g1, d1, sg1, sd1))

    def start(slot):
        g, d, sg, sd = bufs[slot]
        pltpu.async_copy(G1.at[src32.at[slot]], g, sg)
        pltpu.async_copy(AD1.at[dst32.at[slot]], d, sd)

    def wait(slot):
        g, d, sg, sd = bufs[slot]
        pltpu.make_async_copy(G1.at[src32.at[slot]], g, sg).wait()
        pltpu.make_async_copy(AD1.at[dst32.at[slot]], d, sd).wait()

    def compute(slot):
        g, d, _, _ = bufs[slot]
        m8 = lax.iota(jnp.int32, 16) < 8

        def edge(e, c2):
            ev = g[e, pl.ds(HC, 16)] + d[e, pl.ds(0, 16)]
            ev = jnp.where(ev > 0, ev, NEG * ev)
            w16 = jnp.exp(ev)
            mw[e, pl.ds(HC, 16)] = w16
            for k in range(4):
                sl = pl.ds(16 * k, 16)
                wa = jnp.full((16,), w16[2 * k], jnp.float32)
                wb = jnp.full((16,), w16[2 * k + 1], jnp.float32)
                wsel = jnp.where(m8, wa, wb)
                mw[e, sl] = wsel * g[e, sl]
            return c2

        lax.fori_loop(0, BL, edge, 0, unroll=False)
        pltpu.sync_copy(mw, acc.at[dst32.at[slot]], add=True)

    _unpack_idx(s16, d16, src32, dst32, 0, 0)
    start(0)

    def two_chunks(j2, carry):
        ja = 2 * j2
        _unpack_idx(s16, d16, src32, dst32, ja + 1, 1)
        start(1)
        wait(0)
        compute(0)
        _unpack_idx(s16, d16, src32, dst32, ja + 2, 0)
        start(0)
        wait(1)
        compute(1)
        return carry

    lax.fori_loop(0, NCHUNKL // 2, two_chunks, 0, unroll=False)
    wait(0)
    plsc.subcore_barrier()
    base = cid * NP_ + r0
    pltpu.sync_copy(acc.at[pl.ds(r0, RPS)], out_acc.at[pl.ds(base, RPS)])


def _sc1(G1a, AD1a, srcs16, dsts16):
    mesh = plsc.VectorSubcoreMesh(core_axis_name="c", subcore_axis_name="s")
    f = pl.kernel(
        _sc1_body,
        out_type=jax.ShapeDtypeStruct((2 * NP_, G1W), jnp.float32),
        mesh=mesh,
        compiler_params=_SC_PARAMS,
        scratch_types=[
            pltpu.VMEM((NCH2L, BL), jnp.int16),
            pltpu.VMEM((NCH2L, BL), jnp.int16),
            pltpu.VMEM((2, BL), jnp.int32),
            pltpu.VMEM((2, BL), jnp.int32),
            pltpu.VMEM((BL, G1W), jnp.float32),
            pltpu.VMEM((BL, LW), jnp.float32),
            pltpu.VMEM((BL, G1W), jnp.float32),
            pltpu.VMEM((BL, LW), jnp.float32),
            pltpu.VMEM((BL, G1W), jnp.float32),
            pltpu.VMEM_SHARED((NP_, G1W), jnp.float32),
            pltpu.SemaphoreType.DMA,
            pltpu.SemaphoreType.DMA,
            pltpu.SemaphoreType.DMA,
            pltpu.SemaphoreType.DMA,
        ],
    )
    z80 = jnp.zeros((NP_, G1W), jnp.float32)
    return f(G1a, AD1a, srcs16, dsts16, z80)


def _sc2a_body(AS2, AD2, srcs16, dsts16, z16,
               out_den,
               s16, d16, src32, dst32, s0, d0, s1, d1, w_c,
               acc, ss0, sd0, ss1, sd1):
    cid = lax.axis_index("c")
    sid = lax.axis_index("s")
    wid = cid * 16 + sid

    _stage_idx(srcs16, dsts16, wid, s16, d16)
    r0 = sid * RPS
    pltpu.sync_copy(z16.at[pl.ds(r0, RPS)], acc.at[pl.ds(r0, RPS)])
    plsc.subcore_barrier()

    bufs = ((s0, d0, ss0, sd0), (s1, d1, ss1, sd1))

    def start(slot):
        s, d, ss, sd = bufs[slot]
        pltpu.async_copy(AS2.at[src32.at[slot]], s, ss)
        pltpu.async_copy(AD2.at[dst32.at[slot]], d, sd)

    def wait(slot):
        s, d, ss, sd = bufs[slot]
        pltpu.make_async_copy(AS2.at[src32.at[slot]], s, ss).wait()
        pltpu.make_async_copy(AD2.at[dst32.at[slot]], d, sd).wait()

    def compute(slot):
        s, d, _, _ = bufs[slot]

        def edge(e, c2):
            ev = s[e, pl.ds(0, 16)] + d[e, pl.ds(0, 16)]
            ev = jnp.where(ev > 0, ev, NEG * ev)
            w_c[e, pl.ds(0, 16)] = jnp.exp(ev)
            return c2

        lax.fori_loop(0, BL, edge, 0, unroll=False)
        pltpu.sync_copy(w_c, acc.at[dst32.at[slot]], add=True)

    _unpack_idx(s16, d16, src32, dst32, 0, 0)
    start(0)

    def two_chunks(j2, carry):
        ja = 2 * j2
        _unpack_idx(s16, d16, src32, dst32, ja + 1, 1)
        start(1)
        wait(0)
        compute(0)
        _unpack_idx(s16, d16, src32, dst32, ja + 2, 0)
        start(0)
        wait(1)
        compute(1)
        return carry

    lax.fori_loop(0, NCHUNKL // 2, two_chunks, 0, unroll=False)
    wait(0)
    plsc.subcore_barrier()
    base = cid * NP_ + r0
    pltpu.sync_copy(acc.at[pl.ds(r0, RPS)], out_den.at[pl.ds(base, RPS)])


def _sc2a(AS2, AD2, srcs16, dsts16):
    mesh = plsc.VectorSubcoreMesh(core_axis_name="c", subcore_axis_name="s")
    f = pl.kernel(
        _sc2a_body,
        out_type=jax.ShapeDtypeStruct((2 * NP_, LW), jnp.float32),
        mesh=mesh,
        compiler_params=_SC_PARAMS,
        scratch_types=[
            pltpu.VMEM((NCH2L, BL), jnp.int16),
            pltpu.VMEM((NCH2L, BL), jnp.int16),
            pltpu.VMEM((2, BL), jnp.int32),
            pltpu.VMEM((2, BL), jnp.int32),
            pltpu.VMEM((BL, LW), jnp.float32),
            pltpu.VMEM((BL, LW), jnp.float32),
            pltpu.VMEM((BL, LW), jnp.float32),
            pltpu.VMEM((BL, LW), jnp.float32),
            pltpu.VMEM((BL, LW), jnp.float32),
            pltpu.VMEM_SHARED((NP_, LW), jnp.float32),
            pltpu.SemaphoreType.DMA,
            pltpu.SemaphoreType.DMA,
            pltpu.SemaphoreType.DMA,
            pltpu.SemaphoreType.DMA,
        ],
    )
    z16 = jnp.zeros((NP_, LW), jnp.float32)
    return f(AS2, AD2, srcs16, dsts16, z16)


def _sc2b_body(G2, D2, srcs16, dsts16, z48,
               out_num,
               s16, d16, src32, dst32, g0, d0, g1, d1, msg,
               acc, sg0, sd0, sg1, sd1):
    cid = lax.axis_index("c")
    sid = lax.axis_index("s")
    wid = cid * 16 + sid

    _stage_idx(srcs16, dsts16, wid, s16, d16)
    r0 = sid * RPS
    pltpu.sync_copy(z48.at[pl.ds(r0, RPS)], acc.at[pl.ds(r0, RPS)])
    plsc.subcore_barrier()

    bufs = ((g0, d0, sg0, sd0), (g1, d1, sg1, sd1))

    def start(slot):
        g, d, sg, sd = bufs[slot]
        pltpu.async_copy(G2.at[src32.at[slot]], g, sg)
        pltpu.async_copy(D2.at[dst32.at[slot]], d, sd)

    def wait(slot):
        g, d, sg, sd = bufs[slot]
        pltpu.make_async_copy(G2.at[src32.at[slot]], g, sg).wait()
        pltpu.make_async_copy(D2.at[dst32.at[slot]], d, sd).wait()

    def compute(slot):
        g, d, _, _ = bufs[slot]

        def edge(e, c2):
            ev = g[e, pl.ds(HNCP, 16)] + d[e, pl.ds(0, 16)]
            ev = jnp.where(ev > 0, ev, NEG * ev)
            b16 = jnp.exp(ev) * d[e, pl.ds(LW, 16)]
            a0 = jnp.zeros((16,), jnp.float32)
            a1 = jnp.zeros((16,), jnp.float32)
            a2 = jnp.zeros((16,), jnp.float32)
            for h in range(H):
                wb = jnp.full((16,), b16[h], jnp.float32)
                a0 = a0 + wb * g[e, pl.ds(h * NCP, 16)]
                a1 = a1 + wb * g[e, pl.ds(h * NCP + 16, 16)]
                a2 = a2 + wb * g[e, pl.ds(h * NCP + 32, 16)]
            msg[e, pl.ds(0, 16)] = a0
            msg[e, pl.ds(16, 16)] = a1
            msg[e, pl.ds(32, 16)] = a2
            return c2

        lax.fori_loop(0, B, edge, 0, unroll=False)
        pltpu.sync_copy(msg, acc.at[dst32.at[slot]], add=True)

    _unpack_idx(s16, d16, src32, dst32, 0, 0)
    start(0)

    def two_chunks(j2, carry):
        ja = 2 * j2
        _unpack_idx(s16, d16, src32, dst32, ja + 1, 1)
        start(1)
        wait(0)
        compute(0)
        _unpack_idx(s16, d16, src32, dst32, ja + 2, 0)
        start(0)
        wait(1)
        compute(1)
        return carry

    lax.fori_loop(0, NCHUNK // 2, two_chunks, 0, unroll=False)
    wait(0)
    plsc.subcore_barrier()
    base = cid * NP_ + r0
    pltpu.sync_copy(acc.at[pl.ds(r0, RPS)], out_num.at[pl.ds(base, RPS)])


def _sc2b(G2a, D2a, srcs16, dsts16):
    mesh = plsc.VectorSubcoreMesh(core_axis_name="c", subcore_axis_name="s")
    f = pl.kernel(
        _sc2b_body,
        out_type=jax.ShapeDtypeStruct((2 * NP_, NCP), jnp.float32),
        mesh=mesh,
        compiler_params=_SC_PARAMS,
        scratch_types=[
            pltpu.VMEM((NCH2, B), jnp.int16),
            pltpu.VMEM((NCH2, B), jnp.int16),
            pltpu.VMEM((2, B), jnp.int32),
            pltpu.VMEM((2, B), jnp.int32),
            pltpu.VMEM((B, G2W), jnp.float32),
            pltpu.VMEM((B, D2W), jnp.float32),
            pltpu.VMEM((B, G2W), jnp.float32),
            pltpu.VMEM((B, D2W), jnp.float32),
            pltpu.VMEM((B, NCP), jnp.float32),
            pltpu.VMEM_SHARED((NP_, NCP), jnp.float32),
            pltpu.SemaphoreType.DMA,
            pltpu.SemaphoreType.DMA,
            pltpu.SemaphoreType.DMA,
            pltpu.SemaphoreType.DMA,
        ],
    )
    z48 = jnp.zeros((NP_, NCP), jnp.float32)
    return f(G2a, D2a, srcs16, dsts16, z48)


def _pack_idx16(arr):
    nw, nch, w = arr.shape
    a = arr.reshape(nw, nch, w // 32, 2, 16)
    a = a.transpose(0, 1, 2, 4, 3)
    return a.reshape(nw, nch, w).astype(jnp.int16)


def kernel(x, edge_index, W1, att_src1, att_dst1, b1, W2, att_src2,
           att_dst2, b2):
    loop = jnp.arange(N, dtype=jnp.int32)
    fill = jnp.full((EP - E - N,), N, jnp.int32)
    src = jnp.concatenate([edge_index[0].astype(jnp.int32), loop, fill])
    dst = jnp.concatenate([edge_index[1].astype(jnp.int32), loop, fill])
    srcs = jnp.pad(src.reshape(NW, NCHUNK, B), ((0, 0), (0, 2), (0, 0)),
                   constant_values=N)
    dsts = jnp.pad(dst.reshape(NW, NCHUNK, B), ((0, 0), (0, 2), (0, 0)),
                   constant_values=N)
    srcs16 = _pack_idx16(srcs)
    dsts16 = _pack_idx16(dsts)
    srcsL = jnp.pad(src.reshape(NW, NCHUNKL, BL), ((0, 0), (0, 2), (0, 0)),
                    constant_values=N)
    dstsL = jnp.pad(dst.reshape(NW, NCHUNKL, BL), ((0, 0), (0, 2), (0, 0)),
                    constant_values=N)
    srcs16L = _pack_idx16(srcsL)
    dsts16L = _pack_idx16(dstsL)

    xp = jnp.pad(x, ((0, NP_ - N), (0, 0)))

    r64 = jnp.arange(HC)
    k16 = jnp.arange(LW)
    head_of_row = r64 // C1
    As1 = jnp.where(head_of_row[:, None] == k16[None, :],
                    att_src1.reshape(-1)[:, None], 0.0)
    Ad1 = jnp.where(head_of_row[:, None] == k16[None, :],
                    att_dst1.reshape(-1)[:, None], 0.0)
    E16 = (k16[:, None] == (r64[None, :] // C1)).astype(jnp.float32)

    r384 = jnp.arange(HNCP)
    rh = r384 // NCP
    rc = r384 % NCP
    att2s_flat = jnp.where(rc < NC, att_src2[rh, jnp.clip(rc, 0, NC - 1)], 0.0)
    att2d_flat = jnp.where(rc < NC, att_dst2[rh, jnp.clip(rc, 0, NC - 1)], 0.0)
    As2M = jnp.where(rh[:, None] == k16[None, :], att2s_flat[:, None], 0.0)
    Ad2M = jnp.where(rh[:, None] == k16[None, :], att2d_flat[:, None], 0.0)

    cols = jnp.arange(HNCP)
    chead = cols // NCP
    coff = cols % NCP
    W2p = jnp.where(coff < NC,
                    W2[:, jnp.clip(chead * NC + coff, 0, H * NC - 1)], 0.0)
    b1r = b1.reshape(1, HC)
    b2p = jnp.pad(b2, (0, NCP - NC)).reshape(1, NCP)

    G1a, AD1a = _tc1(xp, W1, As1, Ad1)
    acc1 = _sc1(G1a, AD1a, srcs16L, dsts16L).reshape(2, NP_, G1W)
    G2a, as2s, ad2s = _tc2(acc1, E16, b1r, W2p, As2M, Ad2M)
    den2 = _sc2a(as2s, ad2s, srcs16L, dsts16L)
    D2a = _tc2c(den2.reshape(2, NP_, LW), ad2s)
    num2f = _sc2b(G2a, D2a, srcs16, dsts16)
    num2 = num2f.reshape(2, NP_, NCP)
    out = _tc3(num2, b2p)
    return out[:N, :NC]


# speedup vs baseline: 1.0298x; 1.0298x over previous
import jax
import jax.numpy as jnp
from jax import lax
from jax.experimental import pallas as pl
from jax.experimental.pallas import tpu as pltpu
from jax.experimental.pallas import tpu_sc as plsc

N = 10000
E = 320000
F_IN = 128
H = 8
C1 = 8
HC = H * C1
LW = 16
G1W = HC + LW
NC = 40
NCP = 48
HNCP = H * NCP
NCPB = 64
HNCPB = H * NCPB
G2BW = HNCPB + 2 * LW
MW2 = 64
D2W = 2 * LW
NEG = 0.2

NP_ = 10240
NW = 32
NCHUNK = 168
NCH2 = NCHUNK + 2
B = 64
BL = 128
NCHUNKL = 84
NCH2L = NCHUNKL + 2
EP = NW * NCHUNK * B
RPS = NP_ // 16
BLK = 1024
GRID = NP_ // BLK

_SC_PARAMS = pltpu.CompilerParams(use_tc_tiling_on_sc=False,
                                  needs_layout_passes=False)


def _tc1_body(x_ref, w1_ref, as_ref, ad_ref, g_out, ad_out):
    h = jnp.dot(x_ref[...], w1_ref[...], preferred_element_type=jnp.float32)
    as16 = jnp.dot(h, as_ref[...], preferred_element_type=jnp.float32)
    g_out[...] = jnp.concatenate([h, as16], axis=1)
    ad_out[...] = jnp.dot(h, ad_ref[...], preferred_element_type=jnp.float32)


def _tc1(xp, W1, As1, Ad1):
    return pl.pallas_call(
        _tc1_body,
        grid=(GRID,),
        in_specs=[
            pl.BlockSpec((BLK, F_IN), lambda i: (i, 0)),
            pl.BlockSpec((F_IN, HC), lambda i: (0, 0)),
            pl.BlockSpec((HC, LW), lambda i: (0, 0)),
            pl.BlockSpec((HC, LW), lambda i: (0, 0)),
        ],
        out_specs=[
            pl.BlockSpec((BLK, G1W), lambda i: (i, 0)),
            pl.BlockSpec((BLK, LW), lambda i: (i, 0)),
        ],
        out_shape=[
            jax.ShapeDtypeStruct((NP_, G1W), jnp.float32),
            jax.ShapeDtypeStruct((NP_, LW), jnp.float32),
        ],
    )(xp, W1, As1, Ad1)


def _tc2_body(acc_ref, e16_ref, b1_ref, w2p_ref, as2_ref, ad2_ref,
              as32_ref, g_out, as_out, ad_out):
    num = acc_ref[0][:, :HC] + acc_ref[1][:, :HC]
    den16 = acc_ref[0][:, HC:] + acc_ref[1][:, HC:]
    den = jnp.dot(den16, e16_ref[...], preferred_element_type=jnp.float32)
    out1 = num / (den + 1e-16)
    v = out1 + b1_ref[...]
    h1e = jnp.where(v > 0, v, jnp.exp(v) - 1.0)
    h2 = jnp.dot(h1e, w2p_ref[...], preferred_element_type=jnp.float32)
    as32 = jnp.dot(h2, as32_ref[...], preferred_element_type=jnp.float32)
    g_out[...] = jnp.concatenate([h2, as32], axis=1).astype(jnp.bfloat16)
    as_out[...] = jnp.dot(h2, as2_ref[...], preferred_element_type=jnp.float32)
    ad_out[...] = jnp.dot(h2, ad2_ref[...], preferred_element_type=jnp.float32)


def _tc2(acc1, E16, b1r, W2p, As2M, Ad2M, As2M32):
    return pl.pallas_call(
        _tc2_body,
        grid=(GRID,),
        in_specs=[
            pl.BlockSpec((2, BLK, G1W), lambda i: (0, i, 0)),
            pl.BlockSpec((LW, HC), lambda i: (0, 0)),
            pl.BlockSpec((1, HC), lambda i: (0, 0)),
            pl.BlockSpec((HC, HNCPB), lambda i: (0, 0)),
            pl.BlockSpec((HNCPB, LW), lambda i: (0, 0)),
            pl.BlockSpec((HNCPB, LW), lambda i: (0, 0)),
            pl.BlockSpec((HNCPB, 2 * LW), lambda i: (0, 0)),
        ],
        out_specs=[
            pl.BlockSpec((BLK, G2BW), lambda i: (i, 0)),
            pl.BlockSpec((BLK, LW), lambda i: (i, 0)),
            pl.BlockSpec((BLK, LW), lambda i: (i, 0)),
        ],
        out_shape=[
            jax.ShapeDtypeStruct((NP_, G2BW), jnp.bfloat16),
            jax.ShapeDtypeStruct((NP_, LW), jnp.float32),
            jax.ShapeDtypeStruct((NP_, LW), jnp.float32),
        ],
    )(acc1, E16, b1r, W2p, As2M, Ad2M, As2M32)


def _tc2c_body(den_ref, ad_ref, d_out):
    r = (1.0 / H) / (den_ref[0] + den_ref[1] + 1e-16)
    d_out[...] = jnp.concatenate([ad_ref[...], r], axis=1)


def _tc2c(den2p, ad2s):
    return pl.pallas_call(
        _tc2c_body,
        grid=(GRID,),
        in_specs=[
            pl.BlockSpec((2, BLK, LW), lambda i: (0, i, 0)),
            pl.BlockSpec((BLK, LW), lambda i: (i, 0)),
        ],
        out_specs=pl.BlockSpec((BLK, D2W), lambda i: (i, 0)),
        out_shape=jax.ShapeDtypeStruct((NP_, D2W), jnp.float32),
    )(den2p, ad2s)


def _tc3_body(num_ref, b2_ref, out_ref):
    z = num_ref[0] + num_ref[1] + b2_ref[...]
    col = lax.broadcasted_iota(jnp.int32, (BLK, MW2), 1)
    mask = col < NC
    zm = jnp.where(mask, z, -1e30)
    m = jnp.max(zm, axis=1, keepdims=True)
    s = jnp.sum(jnp.where(mask, jnp.exp(zm - m), 0.0), axis=1, keepdims=True)
    out_ref[...] = (z - m) - jnp.log(s)


def _tc3(num2, b2p):
    return pl.pallas_call(
        _tc3_body,
        grid=(GRID,),
        in_specs=[
            pl.BlockSpec((2, BLK, MW2), lambda i: (0, i, 0)),
            pl.BlockSpec((1, MW2), lambda i: (0, 0)),
        ],
        out_specs=pl.BlockSpec((BLK, MW2), lambda i: (i, 0)),
        out_shape=jax.ShapeDtypeStruct((NP_, MW2), jnp.float32),
    )(num2, b2p)


def _stage_idx(srcs16, dsts16, wid, s16, d16):
    pltpu.sync_copy(srcs16.at[wid], s16)
    pltpu.sync_copy(dsts16.at[wid], d16)


def _unpack_idx(s16, d16, src32, dst32, j, slot):
    for ref16, ref32 in ((s16, src32), (d16, dst32)):
        for t in range(ref16.shape[1] // 32):
            va, vb = plsc.unpack(
                ref16[j, pl.ds(32 * t, 32)],
                format=plsc.PackFormat.INTERLEAVED,
                preferred_element_type=jnp.int32,
            )
            ref32[slot, pl.ds(32 * t, 16)] = va
            ref32[slot, pl.ds(32 * t + 16, 16)] = vb


def _sc1_body(G1, AD1, srcs16, dsts16, z80,
              out_acc,
              s16, d16, src32, dst32, g0, d0, g1, d1, mw,
              acc, sg0, sd0, sg1, sd1):
    cid = lax.axis_index("c")
    sid = lax.axis_index("s")
    wid = cid * 16 + sid

    _stage_idx(srcs16, dsts16, wid, s16, d16)
    r0 = sid * RPS
    pltpu.sync_copy(z80.at[pl.ds(r0, RPS)], acc.at[pl.ds(r0, RPS)])
    plsc.subcore_barrier()

    bufs = ((g0, d0, sg0, sd0), (g1, d1, sg1, sd1))

    def start(slot):
        g, d, sg, sd = bufs[slot]
        pltpu.async_copy(G1.at[src32.at[slot]], g, sg)
        pltpu.async_copy(AD1.at[dst32.at[slot]], d, sd)

    def wait(slot):
        g, d, sg, sd = bufs[slot]
        pltpu.make_async_copy(G1.at[src32.at[slot]], g, sg).wait()
        pltpu.make_async_copy(AD1.at[dst32.at[slot]], d, sd).wait()

    def compute(slot):
        g, d, _, _ = bufs[slot]
        m8 = lax.iota(jnp.int32, 16) < 8

        def edge(e, c2):
            ev = g[e, pl.ds(HC, 16)] + d[e, pl.ds(0, 16)]
            ev = jnp.where(ev > 0, ev, NEG * ev)
            w16 = jnp.exp(ev)
            mw[e, pl.ds(HC, 16)] = w16
            for k in range(4):
                sl = pl.ds(16 * k, 16)
                wa = jnp.full((16,), w16[2 * k], jnp.float32)
                wb = jnp.full((16,), w16[2 * k + 1], jnp.float32)
                wsel = jnp.where(m8, wa, wb)
                mw[e, sl] = wsel * g[e, sl]
            return c2

        lax.fori_loop(0, BL, edge, 0, unroll=False)
        pltpu.sync_copy(mw, acc.at[dst32.at[slot]], add=True)

    _unpack_idx(s16, d16, src32, dst32, 0, 0)
    start(0)

    def two_chunks(j2, carry):
        ja = 2 * j2
        _unpack_idx(s16, d16, src32, dst32, ja + 1, 1)
        start(1)
        wait(0)
        compute(0)
        _unpack_idx(s16, d16, src32, dst32, ja + 2, 0)
        start(0)
        wait(1)
        compute(1)
        return carry

    lax.fori_loop(0, NCHUNKL // 2, two_chunks, 0, unroll=False)
    wait(0)
    plsc.subcore_barrier()
    base = cid * NP_ + r0
    pltpu.sync_copy(acc.at[pl.ds(r0, RPS)], out_acc.at[pl.ds(base, RPS)])


def _sc1(G1a, AD1a, srcs16, dsts16):
    mesh = plsc.VectorSubcoreMesh(core_axis_name="c", subcore_axis_name="s")
    f = pl.kernel(
        _sc1_body,
        out_type=jax.ShapeDtypeStruct((2 * NP_, G1W), jnp.float32),
        mesh=mesh,
        compiler_params=_SC_PARAMS,
        scratch_types=[
            pltpu.VMEM((NCH2L, BL), jnp.int16),
            pltpu.VMEM((NCH2L, BL), jnp.int16),
            pltpu.VMEM((2, BL), jnp.int32),
            pltpu.VMEM((2, BL), jnp.int32),
            pltpu.VMEM((BL, G1W), jnp.float32),
            pltpu.VMEM((BL, LW), jnp.float32),
            pltpu.VMEM((BL, G1W), jnp.float32),
            pltpu.VMEM((BL, LW), jnp.float32),
            pltpu.VMEM((BL, G1W), jnp.float32),
            pltpu.VMEM_SHARED((NP_, G1W), jnp.float32),
            pltpu.SemaphoreType.DMA,
            pltpu.SemaphoreType.DMA,
            pltpu.SemaphoreType.DMA,
            pltpu.SemaphoreType.DMA,
        ],
    )
    z80 = jnp.zeros((NP_, G1W), jnp.float32)
    return f(G1a, AD1a, srcs16, dsts16, z80)


def _sc2a_body(AS2, AD2, srcs16, dsts16, z16,
               out_den,
               s16, d16, src32, dst32, s0, d0, s1, d1, w_c,
               acc, ss0, sd0, ss1, sd1):
    cid = lax.axis_index("c")
    sid = lax.axis_index("s")
    wid = cid * 16 + sid

    _stage_idx(srcs16, dsts16, wid, s16, d16)
    r0 = sid * RPS
    pltpu.sync_copy(z16.at[pl.ds(r0, RPS)], acc.at[pl.ds(r0, RPS)])
    plsc.subcore_barrier()

    bufs = ((s0, d0, ss0, sd0), (s1, d1, ss1, sd1))

    def start(slot):
        s, d, ss, sd = bufs[slot]
        pltpu.async_copy(AS2.at[src32.at[slot]], s, ss)
        pltpu.async_copy(AD2.at[dst32.at[slot]], d, sd)

    def wait(slot):
        s, d, ss, sd = bufs[slot]
        pltpu.make_async_copy(AS2.at[src32.at[slot]], s, ss).wait()
        pltpu.make_async_copy(AD2.at[dst32.at[slot]], d, sd).wait()

    def compute(slot):
        s, d, _, _ = bufs[slot]

        def edge(e, c2):
            ev = s[e, pl.ds(0, 16)] + d[e, pl.ds(0, 16)]
            ev = jnp.where(ev > 0, ev, NEG * ev)
            w_c[e, pl.ds(0, 16)] = jnp.exp(ev)
            return c2

        lax.fori_loop(0, BL, edge, 0, unroll=False)
        pltpu.sync_copy(w_c, acc.at[dst32.at[slot]], add=True)

    _unpack_idx(s16, d16, src32, dst32, 0, 0)
    start(0)

    def two_chunks(j2, carry):
        ja = 2 * j2
        _unpack_idx(s16, d16, src32, dst32, ja + 1, 1)
        start(1)
        wait(0)
        compute(0)
        _unpack_idx(s16, d16, src32, dst32, ja + 2, 0)
        start(0)
        wait(1)
        compute(1)
        return carry

    lax.fori_loop(0, NCHUNKL // 2, two_chunks, 0, unroll=False)
    wait(0)
    plsc.subcore_barrier()
    base = cid * NP_ + r0
    pltpu.sync_copy(acc.at[pl.ds(r0, RPS)], out_den.at[pl.ds(base, RPS)])


def _sc2a(AS2, AD2, srcs16, dsts16):
    mesh = plsc.VectorSubcoreMesh(core_axis_name="c", subcore_axis_name="s")
    f = pl.kernel(
        _sc2a_body,
        out_type=jax.ShapeDtypeStruct((2 * NP_, LW), jnp.float32),
        mesh=mesh,
        compiler_params=_SC_PARAMS,
        scratch_types=[
            pltpu.VMEM((NCH2L, BL), jnp.int16),
            pltpu.VMEM((NCH2L, BL), jnp.int16),
            pltpu.VMEM((2, BL), jnp.int32),
            pltpu.VMEM((2, BL), jnp.int32),
            pltpu.VMEM((BL, LW), jnp.float32),
            pltpu.VMEM((BL, LW), jnp.float32),
            pltpu.VMEM((BL, LW), jnp.float32),
            pltpu.VMEM((BL, LW), jnp.float32),
            pltpu.VMEM((BL, LW), jnp.float32),
            pltpu.VMEM_SHARED((NP_, LW), jnp.float32),
            pltpu.SemaphoreType.DMA,
            pltpu.SemaphoreType.DMA,
            pltpu.SemaphoreType.DMA,
            pltpu.SemaphoreType.DMA,
        ],
    )
    z16 = jnp.zeros((NP_, LW), jnp.float32)
    return f(AS2, AD2, srcs16, dsts16, z16)


def _sc2b_body(G2, D2, srcs16, dsts16, z48,
               out_num,
               s16, d16, src32, dst32, g0, d0, g1, d1, msg,
               acc, sg0, sd0, sg1, sd1):
    cid = lax.axis_index("c")
    sid = lax.axis_index("s")
    wid = cid * 16 + sid

    _stage_idx(srcs16, dsts16, wid, s16, d16)
    r0 = sid * RPS
    pltpu.sync_copy(z48.at[pl.ds(r0, RPS)], acc.at[pl.ds(r0, RPS)])
    plsc.subcore_barrier()

    bufs = ((g0, d0, sg0, sd0), (g1, d1, sg1, sd1))

    def start(slot):
        g, d, sg, sd = bufs[slot]
        pltpu.async_copy(G2.at[src32.at[slot]], g, sg)
        pltpu.async_copy(D2.at[dst32.at[slot]], d, sd)

    def wait(slot):
        g, d, sg, sd = bufs[slot]
        pltpu.make_async_copy(G2.at[src32.at[slot]], g, sg).wait()
        pltpu.make_async_copy(D2.at[dst32.at[slot]], d, sd).wait()

    def compute(slot):
        g, d, _, _ = bufs[slot]

        def edge(e, c2):
            alo, _ = plsc.unpack(g[e, pl.ds(HNCPB, 32)],
                                 format=plsc.PackFormat.INTERLEAVED,
                                 preferred_element_type=jnp.float32)
            ev = alo + d[e, pl.ds(0, 16)]
            ev = jnp.where(ev > 0, ev, NEG * ev)
            b16 = jnp.exp(ev) * d[e, pl.ds(LW, 16)]
            a0 = jnp.zeros((16,), jnp.float32)
            a1 = jnp.zeros((16,), jnp.float32)
            a2 = jnp.zeros((16,), jnp.float32)
            a3 = jnp.zeros((16,), jnp.float32)
            for h in range(H):
                wb = jnp.full((16,), b16[h], jnp.float32)
                lo0, hi0 = plsc.unpack(g[e, pl.ds(h * NCPB, 32)],
                                       format=plsc.PackFormat.INTERLEAVED,
                                       preferred_element_type=jnp.float32)
                lo1, hi1 = plsc.unpack(g[e, pl.ds(h * NCPB + 32, 32)],
                                       format=plsc.PackFormat.INTERLEAVED,
                                       preferred_element_type=jnp.float32)
                a0 = a0 + wb * lo0
                a1 = a1 + wb * hi0
                a2 = a2 + wb * lo1
                a3 = a3 + wb * hi1
            msg[e, pl.ds(0, 16)] = a0
            msg[e, pl.ds(16, 16)] = a1
            msg[e, pl.ds(32, 16)] = a2
            msg[e, pl.ds(48, 16)] = a3
            return c2

        lax.fori_loop(0, B, edge, 0, unroll=False)
        pltpu.sync_copy(msg, acc.at[dst32.at[slot]], add=True)

    _unpack_idx(s16, d16, src32, dst32, 0, 0)
    start(0)

    def two_chunks(j2, carry):
        ja = 2 * j2
        _unpack_idx(s16, d16, src32, dst32, ja + 1, 1)
        start(1)
        wait(0)
        compute(0)
        _unpack_idx(s16, d16, src32, dst32, ja + 2, 0)
        start(0)
        wait(1)
        compute(1)
        return carry

    lax.fori_loop(0, NCHUNK // 2, two_chunks, 0, unroll=False)
    wait(0)
    plsc.subcore_barrier()
    base = cid * NP_ + r0
    pltpu.sync_copy(acc.at[pl.ds(r0, RPS)], out_num.at[pl.ds(base, RPS)])


def _sc2b(G2a, D2a, srcs16, dsts16):
    mesh = plsc.VectorSubcoreMesh(core_axis_name="c", subcore_axis_name="s")
    f = pl.kernel(
        _sc2b_body,
        out_type=jax.ShapeDtypeStruct((2 * NP_, MW2), jnp.float32),
        mesh=mesh,
        compiler_params=_SC_PARAMS,
        scratch_types=[
            pltpu.VMEM((NCH2, B), jnp.int16),
            pltpu.VMEM((NCH2, B), jnp.int16),
            pltpu.VMEM((2, B), jnp.int32),
            pltpu.VMEM((2, B), jnp.int32),
            pltpu.VMEM((B, G2BW), jnp.bfloat16),
            pltpu.VMEM((B, D2W), jnp.float32),
            pltpu.VMEM((B, G2BW), jnp.bfloat16),
            pltpu.VMEM((B, D2W), jnp.float32),
            pltpu.VMEM((B, MW2), jnp.float32),
            pltpu.VMEM_SHARED((NP_, MW2), jnp.float32),
            pltpu.SemaphoreType.DMA,
            pltpu.SemaphoreType.DMA,
            pltpu.SemaphoreType.DMA,
            pltpu.SemaphoreType.DMA,
        ],
    )
    z48 = jnp.zeros((NP_, MW2), jnp.float32)
    return f(G2a, D2a, srcs16, dsts16, z48)


def _pack_idx16(arr):
    nw, nch, w = arr.shape
    a = arr.reshape(nw, nch, w // 32, 2, 16)
    a = a.transpose(0, 1, 2, 4, 3)
    return a.reshape(nw, nch, w).astype(jnp.int16)


def kernel(x, edge_index, W1, att_src1, att_dst1, b1, W2, att_src2,
           att_dst2, b2):
    loop = jnp.arange(N, dtype=jnp.int32)
    fill = jnp.full((EP - E - N,), N, jnp.int32)
    src = jnp.concatenate([edge_index[0].astype(jnp.int32), loop, fill])
    dst = jnp.concatenate([edge_index[1].astype(jnp.int32), loop, fill])
    srcs = jnp.pad(src.reshape(NW, NCHUNK, B), ((0, 0), (0, 2), (0, 0)),
                   constant_values=N)
    dsts = jnp.pad(dst.reshape(NW, NCHUNK, B), ((0, 0), (0, 2), (0, 0)),
                   constant_values=N)
    srcs16 = _pack_idx16(srcs)
    dsts16 = _pack_idx16(dsts)
    srcsL = jnp.pad(src.reshape(NW, NCHUNKL, BL), ((0, 0), (0, 2), (0, 0)),
                    constant_values=N)
    dstsL = jnp.pad(dst.reshape(NW, NCHUNKL, BL), ((0, 0), (0, 2), (0, 0)),
                    constant_values=N)
    srcs16L = _pack_idx16(srcsL)
    dsts16L = _pack_idx16(dstsL)

    xp = jnp.pad(x, ((0, NP_ - N), (0, 0)))

    r64 = jnp.arange(HC)
    k16 = jnp.arange(LW)
    head_of_row = r64 // C1
    As1 = jnp.where(head_of_row[:, None] == k16[None, :],
                    att_src1.reshape(-1)[:, None], 0.0)
    Ad1 = jnp.where(head_of_row[:, None] == k16[None, :],
                    att_dst1.reshape(-1)[:, None], 0.0)
    E16 = (k16[:, None] == (r64[None, :] // C1)).astype(jnp.float32)

    m512 = jnp.arange(HNCPB)
    mh = m512 // NCPB
    q64 = m512 % NCPB
    mt = q64 // 32
    mq = q64 % 32
    mc = mt * 32 + mq // 2 + (mq % 2) * 16
    real = mc < NC
    W2p = jnp.where(real[None, :],
                    W2[:, jnp.clip(mh * NC + mc, 0, H * NC - 1)], 0.0)
    att2s_m = jnp.where(real, att_src2[mh, jnp.clip(mc, 0, NC - 1)], 0.0)
    att2d_m = jnp.where(real, att_dst2[mh, jnp.clip(mc, 0, NC - 1)], 0.0)
    As2M = jnp.where(mh[:, None] == k16[None, :], att2s_m[:, None], 0.0)
    Ad2M = jnp.where(mh[:, None] == k16[None, :], att2d_m[:, None], 0.0)
    q32 = jnp.arange(2 * LW)
    As2M32 = jnp.where((q32[None, :] % 2 == 0) & (mh[:, None] == q32[None, :] // 2),
                       att2s_m[:, None], 0.0)
    b1r = b1.reshape(1, HC)
    b2p = jnp.pad(b2, (0, MW2 - NC)).reshape(1, MW2)

    G1a, AD1a = _tc1(xp, W1, As1, Ad1)
    acc1 = _sc1(G1a, AD1a, srcs16L, dsts16L).reshape(2, NP_, G1W)
    G2a, as2s, ad2s = _tc2(acc1, E16, b1r, W2p, As2M, Ad2M, As2M32)
    den2 = _sc2a(as2s, ad2s, srcs16L, dsts16L)
    D2a = _tc2c(den2.reshape(2, NP_, LW), ad2s)
    num2f = _sc2b(G2a, D2a, srcs16, dsts16)
    num2 = num2f.reshape(2, NP_, MW2)
    out = _tc3(num2, b2p)
    return out[:N, :NC]
